# Initial kernel scaffold; baseline (speedup 1.0000x reference)
import jax
import jax.numpy as jnp
from jax.experimental import pallas as pl


def kernel(x, edge_index, W, b):
    raise NotImplementedError("write your pallas kernel here")


# speedup vs baseline: 5.8643x; 5.8643x over previous
import functools

import jax
import jax.numpy as jnp
from jax import lax
from jax.experimental import pallas as pl
from jax.experimental.pallas import tpu as pltpu
from jax.experimental.pallas import tpu_sc as plsc

N_NODES = 10000
N_EDGES = 320000
D_FEAT = 128
NUM_CLASSES = 2

_NC, _NS, _L = 2, 16, 16
_NW = _NC * _NS
_EPW = N_EDGES // _NW


def _proj_body(w_ref, x_ref, b_ref, p_ref):
    p_ref[...] = lax.dot_general(
        w_ref[...], x_ref[...],
        dimension_numbers=(((1,), (1,)), ((), ())),
        preferred_element_type=jnp.float32,
    ) + b_ref[...]


_mesh = plsc.VectorSubcoreMesh(core_axis_name="c", subcore_axis_name="s")


@functools.partial(
    pl.kernel,
    mesh=_mesh,
    compiler_params=pltpu.CompilerParams(needs_layout_passes=False),
    out_type=jax.ShapeDtypeStruct((2 * N_EDGES,), jnp.float32),
    scratch_types=[
        pltpu.VMEM((4 * N_NODES,), jnp.float32),
        pltpu.VMEM((_EPW,), jnp.int32),
        pltpu.VMEM((_EPW,), jnp.int32),
        pltpu.VMEM((2 * _EPW,), jnp.float32),
    ],
)
def _edge_score(p_hbm, ei_hbm, out_hbm, p_v, src_v, dst_v, out_v):
    wid = lax.axis_index("s") * _NC + lax.axis_index("c")
    base = wid * _EPW
    pltpu.sync_copy(p_hbm.at[pl.ds(0, 4 * N_NODES)], p_v)
    pltpu.sync_copy(ei_hbm.at[pl.ds(base, _EPW)], src_v)
    pltpu.sync_copy(ei_hbm.at[pl.ds(N_EDGES + base, _EPW)], dst_v)

    def body(i, carry):
        sv = src_v[pl.ds(i * _L, _L)]
        dv = dst_v[pl.ds(i * _L, _L)]
        a0 = plsc.load_gather(p_v, [sv])
        a1 = plsc.load_gather(p_v, [sv + N_NODES])
        c0 = plsc.load_gather(p_v, [dv + 2 * N_NODES])
        c1 = plsc.load_gather(p_v, [dv + 3 * N_NODES])
        pos = i * (2 * _L) + 2 * lax.iota(jnp.int32, _L)
        plsc.store_scatter(out_v, [pos], a0 + c0)
        plsc.store_scatter(out_v, [pos + 1], a1 + c1)
        return carry

    lax.fori_loop(0, _EPW // _L, body, 0)
    pltpu.sync_copy(out_v, out_hbm.at[pl.ds(2 * base, 2 * _EPW)])


def kernel(x, edge_index, W, b):
    W1 = W[:, :D_FEAT]
    W2 = W[:, D_FEAT:]
    Wp = jnp.zeros((8, D_FEAT), jnp.float32).at[:2].set(W1).at[2:4].set(W2)
    bp = jnp.zeros((8, 1), jnp.float32).at[:2, 0].set(b)
    p = pl.pallas_call(
        _proj_body,
        out_shape=jax.ShapeDtypeStruct((8, N_NODES), jnp.float32),
    )(Wp, x, bp)
    ei = edge_index.astype(jnp.int32).reshape(2 * N_EDGES)
    out_flat = _edge_score(p.reshape(8 * N_NODES), ei)
    return out_flat.reshape(N_EDGES, NUM_CLASSES)


# speedup vs baseline: 33.0981x; 5.6440x over previous
import functools

import jax
import jax.numpy as jnp
from jax import lax
from jax.experimental import pallas as pl
from jax.experimental.pallas import tpu as pltpu
from jax.experimental.pallas import tpu_sc as plsc

N_NODES = 10000
N_EDGES = 320000
D_FEAT = 128
NUM_CLASSES = 2

_NC, _NS, _L = 2, 16, 16
_NW = _NC * _NS
_NT = N_EDGES // 128
_TPW = 79
_NPT = 79
_NP = _NPT * 128


def _proj_body(w_ref, x_ref, b_ref, p_ref):
    p_ref[:, pl.ds(0, N_NODES)] = lax.dot_general(
        w_ref[...], x_ref[...],
        dimension_numbers=(((1,), (1,)), ((), ())),
        preferred_element_type=jnp.float32,
    ) + b_ref[...]


_mesh = plsc.VectorSubcoreMesh(core_axis_name="c", subcore_axis_name="s")


@functools.partial(
    pl.kernel,
    mesh=_mesh,
    compiler_params=pltpu.CompilerParams(needs_layout_passes=False),
    out_type=jax.ShapeDtypeStruct((2 * N_EDGES,), jnp.float32),
    scratch_types=[
        pltpu.VMEM((_NPT, 512), jnp.float32),
        pltpu.VMEM((_TPW * 256,), jnp.int32),
        pltpu.VMEM((_TPW * 256,), jnp.float32),
    ],
)
def _edge_score(p_hbm, ei_hbm, out_hbm, p_v, idx_v, out_v):
    wid = lax.axis_index("s") * _NC + lax.axis_index("c")
    tlo = wid * (_NT - _TPW) // (_NW - 1)
    pltpu.sync_copy(p_hbm.at[:, pl.ds(0, 512)], p_v)
    pltpu.sync_copy(ei_hbm.at[pl.ds(tlo * 256, _TPW * 256)], idx_v)

    def body(t, carry):
        base = t * 256
        for g in range(8):
            sv = idx_v[pl.ds(base + 16 * g, 16)]
            dv = idx_v[pl.ds(base + 128 + 16 * g, 16)]
            st, sj = lax.shift_right_logical(sv, 7), sv & 127
            dt, dj = lax.shift_right_logical(dv, 7), dv & 127
            a0 = plsc.load_gather(p_v, [st, sj])
            a1 = plsc.load_gather(p_v, [st, sj + 128])
            c0 = plsc.load_gather(p_v, [dt, dj + 256])
            c1 = plsc.load_gather(p_v, [dt, dj + 384])
            out_v[pl.ds(base + 16 * g, 16)] = a0 + c0
            out_v[pl.ds(base + 128 + 16 * g, 16)] = a1 + c1
        return carry

    lax.fori_loop(0, _TPW, body, 0)
    pltpu.sync_copy(out_v, out_hbm.at[pl.ds(tlo * 256, _TPW * 256)])


def kernel(x, edge_index, W, b):
    W1 = W[:, :D_FEAT]
    W2 = W[:, D_FEAT:]
    Wp = jnp.zeros((8, D_FEAT), jnp.float32).at[:2].set(W1).at[2:4].set(W2)
    bp = jnp.zeros((8, 1), jnp.float32).at[:2, 0].set(b)
    p = pl.pallas_call(
        _proj_body,
        out_shape=jax.ShapeDtypeStruct((8, _NP), jnp.float32),
    )(Wp, x, bp)
    p_tiled = p.reshape(8, _NPT, 128).transpose(1, 0, 2).reshape(_NPT, 1024)
    ei = (edge_index.astype(jnp.int32)
          .reshape(2, _NT, 128).transpose(1, 0, 2).reshape(2 * N_EDGES))
    out_flat = _edge_score(p_tiled, ei)
    return (out_flat.reshape(_NT, 2, 128).transpose(0, 2, 1)
            .reshape(N_EDGES, NUM_CLASSES))


# speedup vs baseline: 40.0802x; 1.2110x over previous
import functools

import jax
import jax.numpy as jnp
from jax import lax
from jax.experimental import pallas as pl
from jax.experimental.pallas import tpu as pltpu
from jax.experimental.pallas import tpu_sc as plsc

N_NODES = 10000
N_EDGES = 320000
D_FEAT = 128
NUM_CLASSES = 2

_NC, _NS, _L = 2, 16, 16
_NW = _NC * _NS
_NT = N_EDGES // 128
_TPW = 79
_NP = 10112


def _proj_body(w_ref, x_ref, b_ref, p_ref):
    p_ref[:, pl.ds(0, N_NODES)] = lax.dot_general(
        w_ref[...], x_ref[...],
        dimension_numbers=(((1,), (1,)), ((), ())),
        preferred_element_type=jnp.float32,
    ) + b_ref[...]


_mesh = plsc.VectorSubcoreMesh(core_axis_name="c", subcore_axis_name="s")


@functools.partial(
    pl.kernel,
    mesh=_mesh,
    compiler_params=pltpu.CompilerParams(needs_layout_passes=False),
    out_type=jax.ShapeDtypeStruct((2 * N_EDGES,), jnp.float32),
    scratch_types=[
        pltpu.VMEM((4, _NP), jnp.float32),
        pltpu.VMEM((_TPW * 128,), jnp.int32),
        pltpu.VMEM((_TPW * 128,), jnp.int32),
        pltpu.VMEM((_TPW * 256,), jnp.float32),
    ],
)
def _edge_score(p_hbm, ei_hbm, out_hbm, p_v, src_v, dst_v, out_v):
    wid = lax.axis_index("s") * _NC + lax.axis_index("c")
    tlo = wid * (_NT - _TPW) // (_NW - 1)
    pltpu.sync_copy(p_hbm.at[pl.ds(0, 4), :], p_v)
    pltpu.sync_copy(ei_hbm.at[0, pl.ds(tlo * 128, _TPW * 128)], src_v)
    pltpu.sync_copy(ei_hbm.at[1, pl.ds(tlo * 128, _TPW * 128)], dst_v)
    r0 = jnp.zeros((16,), jnp.int32)
    r1 = r0 + 1
    r2 = r0 + 2
    r3 = r0 + 3

    def body(t, carry):
        ib = t * 128
        ob = t * 256
        for g in range(8):
            sv = src_v[pl.ds(ib + 16 * g, 16)]
            dv = dst_v[pl.ds(ib + 16 * g, 16)]
            a0 = plsc.load_gather(p_v, [r0, sv])
            a1 = plsc.load_gather(p_v, [r1, sv])
            c0 = plsc.load_gather(p_v, [r2, dv])
            c1 = plsc.load_gather(p_v, [r3, dv])
            out_v[pl.ds(ob + 16 * g, 16)] = a0 + c0
            out_v[pl.ds(ob + 128 + 16 * g, 16)] = a1 + c1
        return carry

    lax.fori_loop(0, _TPW, body, 0)
    pltpu.sync_copy(out_v, out_hbm.at[pl.ds(tlo * 256, _TPW * 256)])


def kernel(x, edge_index, W, b):
    W1 = W[:, :D_FEAT]
    W2 = W[:, D_FEAT:]
    Wp = jnp.zeros((8, D_FEAT), jnp.float32).at[:2].set(W1).at[2:4].set(W2)
    bp = jnp.zeros((8, 1), jnp.float32).at[:2, 0].set(b)
    p = pl.pallas_call(
        _proj_body,
        out_shape=jax.ShapeDtypeStruct((8, _NP), jnp.float32),
    )(Wp, x, bp)
    out_flat = _edge_score(p, edge_index.astype(jnp.int32))
    return (out_flat.reshape(_NT, 2, 128).transpose(0, 2, 1)
            .reshape(N_EDGES, NUM_CLASSES))


# speedup vs baseline: 42.9850x; 1.0725x over previous
import functools

import jax
import jax.numpy as jnp
from jax import lax
from jax.experimental import pallas as pl
from jax.experimental.pallas import tpu as pltpu
from jax.experimental.pallas import tpu_sc as plsc

N_NODES = 10000
N_EDGES = 320000
D_FEAT = 128
NUM_CLASSES = 2

_NC, _NS, _L = 2, 16, 16
_NW = _NC * _NS
_NT = N_EDGES // 128
_TPW = 79
_NP = 10112


def _proj_body(w_ref, x_ref, b_ref, p_ref):
    p_ref[:, pl.ds(0, N_NODES)] = lax.dot_general(
        w_ref[...], x_ref[...],
        dimension_numbers=(((1,), (1,)), ((), ())),
        preferred_element_type=jnp.float32,
    ) + b_ref[...]


_mesh = plsc.VectorSubcoreMesh(core_axis_name="c", subcore_axis_name="s")


@functools.partial(
    pl.kernel,
    mesh=_mesh,
    compiler_params=pltpu.CompilerParams(needs_layout_passes=False),
    out_type=jax.ShapeDtypeStruct((2 * N_EDGES,), jnp.float32),
    scratch_types=[
        pltpu.VMEM((4 * _NP,), jnp.float32),
        pltpu.VMEM((_TPW * 128,), jnp.int32),
        pltpu.VMEM((_TPW * 128,), jnp.int32),
        pltpu.VMEM((_TPW * 256,), jnp.float32),
    ],
)
def _edge_score(p_hbm, ei_hbm, out_hbm, p_v, src_v, dst_v, out_v):
    wid = lax.axis_index("s") * _NC + lax.axis_index("c")
    tlo = wid * (_NT - _TPW) // (_NW - 1)
    for r in range(4):
        pltpu.sync_copy(p_hbm.at[r, :], p_v.at[pl.ds(r * _NP, _NP)])
    pltpu.sync_copy(ei_hbm.at[0, pl.ds(tlo * 128, _TPW * 128)], src_v)
    pltpu.sync_copy(ei_hbm.at[1, pl.ds(tlo * 128, _TPW * 128)], dst_v)
    p_f = p_v

    @plsc.parallel_loop(0, _TPW, unroll=4)
    def body(t):
        ib = t * 128
        ob = t * 256
        for g in range(8):
            sv = src_v[pl.ds(ib + 16 * g, 16)]
            dv = dst_v[pl.ds(ib + 16 * g, 16)]
            a0 = plsc.load_gather(p_f, [sv])
            a1 = plsc.load_gather(p_f, [sv + _NP])
            c0 = plsc.load_gather(p_f, [dv + 2 * _NP])
            c1 = plsc.load_gather(p_f, [dv + 3 * _NP])
            out_v[pl.ds(ob + 16 * g, 16)] = a0 + c0
            out_v[pl.ds(ob + 128 + 16 * g, 16)] = a1 + c1
    pltpu.sync_copy(out_v, out_hbm.at[pl.ds(tlo * 256, _TPW * 256)])


def kernel(x, edge_index, W, b):
    W1 = W[:, :D_FEAT]
    W2 = W[:, D_FEAT:]
    Wp = jnp.zeros((8, D_FEAT), jnp.float32).at[:2].set(W1).at[2:4].set(W2)
    bp = jnp.zeros((8, 1), jnp.float32).at[:2, 0].set(b)
    p = pl.pallas_call(
        _proj_body,
        out_shape=jax.ShapeDtypeStruct((8, _NP), jnp.float32),
    )(Wp, x, bp)
    out_flat = _edge_score(p, edge_index.astype(jnp.int32))
    return (out_flat.reshape(_NT, 2, 128).transpose(0, 2, 1)
            .reshape(N_EDGES, NUM_CLASSES))


# speedup vs baseline: 51.9933x; 1.2096x over previous
import functools

import jax
import jax.numpy as jnp
from jax import lax
from jax.experimental import pallas as pl
from jax.experimental.pallas import tpu as pltpu
from jax.experimental.pallas import tpu_sc as plsc

N_NODES = 10000
N_EDGES = 320000
D_FEAT = 128
NUM_CLASSES = 2

_NC, _NS, _L = 2, 16, 16
_NW = _NC * _NS
_NT = N_EDGES // 128
_TPW = 79
_NP = 10112


def _proj_body(w_ref, x_ref, b_ref, q_ref):
    dims = (((1,), (1,)), ((), ()))
    d1 = lax.dot_general(w_ref[:, :D_FEAT], x_ref[...], dims,
                         preferred_element_type=jnp.float32) + b_ref[...]
    d2 = lax.dot_general(w_ref[:, D_FEAT:], x_ref[...], dims,
                         preferred_element_type=jnp.float32)
    u1 = lax.bitcast_convert_type(
        d1.astype(jnp.bfloat16), jnp.uint16).astype(jnp.int32)
    u2 = lax.bitcast_convert_type(
        d2.astype(jnp.bfloat16), jnp.uint16).astype(jnp.int32)
    q_ref[0:1, pl.ds(0, N_NODES)] = u1[0:1] | (u1[1:2] << 16)
    q_ref[1:2, pl.ds(0, N_NODES)] = u2[0:1] | (u2[1:2] << 16)


_mesh = plsc.VectorSubcoreMesh(core_axis_name="c", subcore_axis_name="s")


@functools.partial(
    pl.kernel,
    mesh=_mesh,
    compiler_params=pltpu.CompilerParams(needs_layout_passes=False),
    out_type=jax.ShapeDtypeStruct((2 * N_EDGES,), jnp.float32),
    scratch_types=[
        pltpu.VMEM((2 * _NP,), jnp.int32),
        pltpu.VMEM((_TPW * 128,), jnp.int32),
        pltpu.VMEM((_TPW * 128,), jnp.int32),
        pltpu.VMEM((_TPW * 256,), jnp.float32),
    ],
)
def _edge_score(q_hbm, ei_hbm, out_hbm, q_v, src_v, dst_v, out_v):
    wid = lax.axis_index("s") * _NC + lax.axis_index("c")
    tlo = wid * (_NT - _TPW) // (_NW - 1)
    for r in range(2):
        pltpu.sync_copy(q_hbm.at[r, :], q_v.at[pl.ds(r * _NP, _NP)])
    pltpu.sync_copy(ei_hbm.at[0, pl.ds(tlo * 128, _TPW * 128)], src_v)
    pltpu.sync_copy(ei_hbm.at[1, pl.ds(tlo * 128, _TPW * 128)], dst_v)

    @plsc.parallel_loop(0, _TPW, unroll=4)
    def body(t):
        ib = t * 128
        ob = t * 256
        for g in range(8):
            sv = src_v[pl.ds(ib + 16 * g, 16)]
            dv = dst_v[pl.ds(ib + 16 * g, 16)]
            ws = plsc.load_gather(q_v, [sv])
            wd = plsc.load_gather(q_v, [dv + _NP])
            a0, a1 = plsc.unpack(plsc.bitcast(ws, jnp.bfloat16),
                                 format=plsc.PackFormat.INTERLEAVED)
            c0, c1 = plsc.unpack(plsc.bitcast(wd, jnp.bfloat16),
                                 format=plsc.PackFormat.INTERLEAVED)
            out_v[pl.ds(ob + 16 * g, 16)] = a0 + c0
            out_v[pl.ds(ob + 128 + 16 * g, 16)] = a1 + c1

    pltpu.sync_copy(out_v, out_hbm.at[pl.ds(tlo * 256, _TPW * 256)])


def kernel(x, edge_index, W, b):
    bp = b.reshape(2, 1)
    q = pl.pallas_call(
        _proj_body,
        out_shape=jax.ShapeDtypeStruct((2, _NP), jnp.int32),
    )(W, x, bp)
    out_flat = _edge_score(q, edge_index.astype(jnp.int32))
    return (out_flat.reshape(_NT, 2, 128).transpose(0, 2, 1)
            .reshape(N_EDGES, NUM_CLASSES))


# speedup vs baseline: 59.5101x; 1.1446x over previous
import functools

import jax
import jax.numpy as jnp
from jax import lax
from jax.experimental import pallas as pl
from jax.experimental.pallas import tpu as pltpu
from jax.experimental.pallas import tpu_sc as plsc

N_NODES = 10000
N_EDGES = 320000
D_FEAT = 128
NUM_CLASSES = 2

_NC, _NS, _L = 2, 16, 16
_NW = _NC * _NS
_NT = N_EDGES // 128
_TPW = 79
_H0 = 40
_H1 = _TPW - _H0
_NP = 10112


def _proj_body(w_ref, x_ref, b_ref, q_ref):
    dims = (((1,), (1,)), ((), ()))
    d1 = lax.dot_general(w_ref[:, :D_FEAT], x_ref[...], dims,
                         preferred_element_type=jnp.float32)
    d2 = lax.dot_general(w_ref[:, D_FEAT:], x_ref[...], dims,
                         preferred_element_type=jnp.float32)

    def _pack16(row):
        return lax.bitcast_convert_type(
            row.astype(jnp.bfloat16), jnp.uint16).astype(jnp.int32)

    s0 = _pack16(d1[0:1] + b_ref[0])
    s1 = _pack16(d1[1:2] + b_ref[1])
    t0 = _pack16(d2[0:1])
    t1 = _pack16(d2[1:2])
    q_ref[0:1, pl.ds(0, N_NODES)] = s0 | (s1 << 16)
    q_ref[1:2, pl.ds(0, N_NODES)] = t0 | (t1 << 16)


_mesh = plsc.VectorSubcoreMesh(core_axis_name="c", subcore_axis_name="s")


@functools.partial(
    pl.kernel,
    mesh=_mesh,
    compiler_params=pltpu.CompilerParams(needs_layout_passes=False),
    out_type=jax.ShapeDtypeStruct((2 * N_EDGES,), jnp.float32),
    scratch_types=[
        pltpu.VMEM((2 * _NP,), jnp.int32),
        pltpu.VMEM((_TPW * 128,), jnp.int32),
        pltpu.VMEM((_TPW * 128,), jnp.int32),
        pltpu.VMEM((_TPW * 256,), jnp.float32),
        pltpu.SemaphoreType.DMA,
        pltpu.SemaphoreType.DMA,
        pltpu.SemaphoreType.DMA,
    ],
)
def _edge_score(q_hbm, ei_hbm, out_hbm, q_v, src_v, dst_v, out_v,
                sem_a, sem_b, sem_o):
    wid = lax.axis_index("s") * _NC + lax.axis_index("c")
    tlo = wid * (_NT - _TPW) // (_NW - 1)
    eb = tlo * 128
    cp = [
        pltpu.async_copy(q_hbm.at[0, :], q_v.at[pl.ds(0, _NP)], sem_a),
        pltpu.async_copy(q_hbm.at[1, :], q_v.at[pl.ds(_NP, _NP)], sem_a),
        pltpu.async_copy(ei_hbm.at[0, pl.ds(eb, _H0 * 128)],
                         src_v.at[pl.ds(0, _H0 * 128)], sem_a),
        pltpu.async_copy(ei_hbm.at[1, pl.ds(eb, _H0 * 128)],
                         dst_v.at[pl.ds(0, _H0 * 128)], sem_a),
    ]
    cp2 = [
        pltpu.async_copy(ei_hbm.at[0, pl.ds(eb + _H0 * 128, _H1 * 128)],
                         src_v.at[pl.ds(_H0 * 128, _H1 * 128)], sem_b),
        pltpu.async_copy(ei_hbm.at[1, pl.ds(eb + _H0 * 128, _H1 * 128)],
                         dst_v.at[pl.ds(_H0 * 128, _H1 * 128)], sem_b),
    ]
    for c in cp:
        c.wait()

    def make_body(t):
        ib = t * 128
        ob = t * 256
        for g in range(8):
            sv = src_v[pl.ds(ib + 16 * g, 16)]
            dv = dst_v[pl.ds(ib + 16 * g, 16)]
            ws = plsc.load_gather(q_v, [sv])
            wd = plsc.load_gather(q_v, [dv + _NP])
            a0, a1 = plsc.unpack(plsc.bitcast(ws, jnp.bfloat16),
                                 format=plsc.PackFormat.INTERLEAVED)
            c0, c1 = plsc.unpack(plsc.bitcast(wd, jnp.bfloat16),
                                 format=plsc.PackFormat.INTERLEAVED)
            out_v[pl.ds(ob + 16 * g, 16)] = a0 + c0
            out_v[pl.ds(ob + 128 + 16 * g, 16)] = a1 + c1

    plsc.parallel_loop(0, _H0, unroll=4)(make_body)
    out0 = pltpu.async_copy(out_v.at[pl.ds(0, _H0 * 256)],
                            out_hbm.at[pl.ds(tlo * 256, _H0 * 256)], sem_o)
    for c in cp2:
        c.wait()
    plsc.parallel_loop(_H0, _TPW, unroll=4)(make_body)
    out1 = pltpu.async_copy(out_v.at[pl.ds(_H0 * 256, _H1 * 256)],
                            out_hbm.at[pl.ds((tlo + _H0) * 256, _H1 * 256)],
                            sem_o)
    out0.wait()
    out1.wait()


def kernel(x, edge_index, W, b):
    q = pl.pallas_call(
        _proj_body,
        in_specs=[
            pl.BlockSpec(memory_space=pltpu.VMEM),
            pl.BlockSpec(memory_space=pltpu.VMEM),
            pl.BlockSpec(memory_space=pltpu.SMEM),
        ],
        out_shape=jax.ShapeDtypeStruct((2, _NP), jnp.int32),
    )(W, x, b)
    out_flat = _edge_score(q, edge_index.astype(jnp.int32))
    return (out_flat.reshape(_NT, 2, 128).transpose(0, 2, 1)
            .reshape(N_EDGES, NUM_CLASSES))
